# baseline (speedup 1.0000x reference)
import functools

import jax
import jax.numpy as jnp
from jax import lax
from jax.experimental import pallas as pl
from jax.experimental.pallas import tpu as pltpu
from jax.experimental.pallas import tpu_sc as plsc

NUM_EMB = 1000000
DIM = 64
B_TOKENS = 4096
SEQ = 200
B = B_TOKENS * SEQ

_info = plsc.get_sparse_core_info()
NC, NS = _info.num_cores, _info.num_subcores
NW = NC * NS
B_PER_W = B // NW
CHUNK = 800
NCHUNKS = B_PER_W // CHUNK


@functools.partial(
    pl.kernel,
    mesh=plsc.VectorSubcoreMesh(core_axis_name="c", subcore_axis_name="s"),
    out_type=jax.ShapeDtypeStruct((B, DIM), jnp.float32),
    scratch_types=[
        pltpu.VMEM((CHUNK,), jnp.int32),
        pltpu.VMEM((CHUNK, DIM), jnp.float32),
        pltpu.SemaphoreType.DMA,
    ],
    compiler_params=pltpu.CompilerParams(use_tc_tiling_on_sc=False),
)
def _gather_kernel(tok_hbm, table_hbm, out_hbm, idx_v, rows_v, sem):
    wid = lax.axis_index("s") * NC + lax.axis_index("c")
    base = wid * B_PER_W

    def chunk_body(g, carry):
        off = base + g * CHUNK
        pltpu.sync_copy(tok_hbm.at[pl.ds(off, CHUNK)], idx_v)
        pltpu.async_copy(table_hbm.at[idx_v], rows_v, sem).wait()
        pltpu.sync_copy(rows_v, out_hbm.at[pl.ds(off, CHUNK)])
        return carry

    lax.fori_loop(0, NCHUNKS, chunk_body, 0)


def kernel(token_ids, emb_mat):
    flat_ids = token_ids.reshape(B).astype(jnp.int32)
    out = _gather_kernel(flat_ids, emb_mat)
    return out.reshape(B_TOKENS, SEQ, DIM)


# speedup vs baseline: 1.0178x; 1.0178x over previous
import functools

import jax
import jax.numpy as jnp
from jax import lax
from jax.experimental import pallas as pl
from jax.experimental.pallas import tpu as pltpu
from jax.experimental.pallas import tpu_sc as plsc

NUM_EMB = 1000000
DIM = 64
B_TOKENS = 4096
SEQ = 200
B = B_TOKENS * SEQ

_info = plsc.get_sparse_core_info()
NC, NS = _info.num_cores, _info.num_subcores
NW = NC * NS
B_PER_W = B // NW
CHUNK = 800
NCHUNKS = B_PER_W // CHUNK
NPAIR = NCHUNKS // 2


@functools.partial(
    pl.kernel,
    mesh=plsc.VectorSubcoreMesh(core_axis_name="c", subcore_axis_name="s"),
    out_type=jax.ShapeDtypeStruct((B, DIM), jnp.float32),
    scratch_types=[
        pltpu.VMEM((B_PER_W,), jnp.int32),
        pltpu.VMEM((CHUNK, DIM), jnp.float32),
        pltpu.VMEM((CHUNK, DIM), jnp.float32),
        pltpu.SemaphoreType.DMA,
        pltpu.SemaphoreType.DMA,
        pltpu.SemaphoreType.DMA,
        pltpu.SemaphoreType.DMA,
    ],
    compiler_params=pltpu.CompilerParams(use_tc_tiling_on_sc=False),
)
def _gather_kernel(tok_hbm, table_hbm, out_hbm, idx_v, rows0, rows1,
                   gsem0, gsem1, wsem0, wsem1):
    wid = lax.axis_index("s") * NC + lax.axis_index("c")
    base = wid * B_PER_W

    def gather(chunk_id, rows, gsem, issue=True):
        mk = pltpu.async_copy if issue else pltpu.make_async_copy
        return mk(
            table_hbm.at[idx_v.at[pl.ds(chunk_id * CHUNK, CHUNK)]], rows, gsem)

    def write(chunk_id, rows, wsem, issue=True):
        mk = pltpu.async_copy if issue else pltpu.make_async_copy
        return mk(
            rows, out_hbm.at[pl.ds(base + chunk_id * CHUNK, CHUNK)], wsem)

    pltpu.sync_copy(tok_hbm.at[pl.ds(base, B_PER_W)], idx_v)
    gather(0, rows0, gsem0)
    gather(1, rows1, gsem1)

    def pair_body(i, carry):
        g0 = 2 * i
        gather(g0, rows0, gsem0, issue=False).wait()
        write(g0, rows0, wsem0)
        gather(g0 + 1, rows1, gsem1, issue=False).wait()
        write(g0 + 1, rows1, wsem1)

        @pl.when(i < NPAIR - 1)
        def _():
            write(g0, rows0, wsem0, issue=False).wait()
            gather(g0 + 2, rows0, gsem0)
            write(g0 + 1, rows1, wsem1, issue=False).wait()
            gather(g0 + 3, rows1, gsem1)

        return carry

    lax.fori_loop(0, NPAIR, pair_body, 0)
    write(NCHUNKS - 2, rows0, wsem0, issue=False).wait()
    write(NCHUNKS - 1, rows1, wsem1, issue=False).wait()


def kernel(token_ids, emb_mat):
    flat_ids = token_ids.reshape(B).astype(jnp.int32)
    out = _gather_kernel(flat_ids, emb_mat)
    return out.reshape(B_TOKENS, SEQ, DIM)
